# Initial kernel scaffold; baseline (speedup 1.0000x reference)
import jax
import jax.numpy as jnp
from jax.experimental import pallas as pl


def kernel(x, edge_index, W_hp, b_hp, W_lp, b_lp, W_i, b_i, w_h, bh, w_l, bl, w_i, bi):
    raise NotImplementedError("write your pallas kernel here")


# speedup vs baseline: 11.2556x; 11.2556x over previous
import functools

import jax
import jax.numpy as jnp
from jax import lax
from jax.experimental import pallas as pl
from jax.experimental.pallas import tpu as pltpu
from jax.experimental.pallas import tpu_sc as plsc

N = 10000
D = 128
NE = 320000
NPAD = 10240
NTILES = 32
EPT_DEG = NE // NTILES
EPT_PROP = NE // 16
CHK = 80
ROWS_PER_TILE = NPAD // 16
ZCHK = 128

_MESH = plsc.VectorSubcoreMesh(core_axis_name="c", subcore_axis_name="s")


@functools.partial(
    pl.kernel,
    out_type=jax.ShapeDtypeStruct((2, NPAD), jnp.float32),
    mesh=_MESH,
    scratch_types=[
        pltpu.VMEM((CHK,), jnp.int32),
        pltpu.VMEM((CHK,), jnp.float32),
        pltpu.VMEM((NPAD // 16,), jnp.float32),
        pltpu.VMEM_SHARED((NPAD,), jnp.float32),
    ],
)
def _deg_kernel(dst_hbm, out_hbm, idx_v, ones_v, z_v, acc):
    c = lax.axis_index("c")
    s = lax.axis_index("s")
    for i in range(CHK // 16):
        ones_v[pl.ds(i * 16, 16)] = jnp.full((16,), 1.0, jnp.float32)
    for i in range((NPAD // 16) // 16):
        z_v[pl.ds(i * 16, 16)] = jnp.zeros((16,), jnp.float32)
    strip = NPAD // 16
    pltpu.sync_copy(z_v, acc.at[pl.ds(s * strip, strip)])
    plsc.subcore_barrier()

    base = (c * 16 + s) * EPT_DEG

    def body(j, _):
        off = base + j * CHK
        pltpu.sync_copy(dst_hbm.at[pl.ds(off, CHK)], idx_v)
        pltpu.sync_copy(ones_v, acc.at[idx_v], add=True)
        return ()

    lax.fori_loop(0, EPT_DEG // CHK, body, ())
    plsc.subcore_barrier()
    pltpu.sync_copy(acc.at[pl.ds(s * strip, strip)],
                    out_hbm.at[c, pl.ds(s * strip, strip)])


@functools.partial(
    pl.kernel,
    out_type=jax.ShapeDtypeStruct((2, NPAD, D), jnp.float32),
    mesh=_MESH,
    scratch_types=[
        pltpu.VMEM((CHK,), jnp.int32),
        pltpu.VMEM((CHK,), jnp.int32),
        pltpu.VMEM((CHK, D), jnp.float32),
        pltpu.VMEM_SHARED((NPAD, D), jnp.float32),
        pltpu.SemaphoreType.DMA,
    ],
)
def _prop_kernel(hs_hbm, src2_hbm, dst_hbm, zrows_hbm, out_hbm,
                 si_v, di_v, rows_v, acc, sem):
    c = lax.axis_index("c")
    s = lax.axis_index("s")
    r0 = s * ROWS_PER_TILE
    for q in range(ROWS_PER_TILE // ZCHK):
        pltpu.sync_copy(zrows_hbm, acc.at[pl.ds(r0 + q * ZCHK, ZCHK)])
    plsc.subcore_barrier()

    base = c * NE + s * EPT_PROP

    def body(j, _):
        off = base + j * CHK
        pltpu.sync_copy(src2_hbm.at[pl.ds(off, CHK)], si_v)
        pltpu.async_copy(hs_hbm.at[si_v], rows_v, sem).wait()
        pltpu.sync_copy(dst_hbm.at[pl.ds((s * EPT_PROP) + j * CHK, CHK)], di_v)
        pltpu.sync_copy(rows_v, acc.at[di_v], add=True)
        return ()

    lax.fori_loop(0, EPT_PROP // CHK, body, ())
    plsc.subcore_barrier()
    pltpu.sync_copy(acc.at[pl.ds(r0, ROWS_PER_TILE)],
                    out_hbm.at[c, pl.ds(r0, ROWS_PER_TILE)])


_RB = 2000


def _mm_body(x_ref, w_ref, b_ref, degt_ref, hhp_ref, hs_ref, hi_ref):
    xb = x_ref[...]
    acc = jnp.dot(xb, w_ref[...], preferred_element_type=jnp.float32)
    acc = acc + b_ref[0:1, :]
    degb = degt_ref[...]
    deg = degb[:, 0:1] + degb[:, 1:2]
    dinv = jnp.where(deg > 0, lax.rsqrt(deg), 0.0)
    h_hp = acc[:, 0:D]
    h_lp = acc[:, D:2 * D]
    h_i = acc[:, 2 * D:3 * D]
    hhp_ref[...] = h_hp
    hs_ref[0] = dinv * h_hp
    hs_ref[1] = dinv * h_lp
    hi_ref[...] = jnp.maximum(h_i, 0.0)


def _mm_call(x, w_cat, b_cat, deg_t):
    return pl.pallas_call(
        _mm_body,
        grid=(N // _RB,),
        in_specs=[
            pl.BlockSpec((_RB, D), lambda i: (i, 0)),
            pl.BlockSpec((D, 3 * D), lambda i: (0, 0)),
            pl.BlockSpec((8, 3 * D), lambda i: (0, 0)),
            pl.BlockSpec((_RB, 2), lambda i: (i, 0)),
        ],
        out_specs=[
            pl.BlockSpec((_RB, D), lambda i: (i, 0)),
            pl.BlockSpec((2, _RB, D), lambda i: (0, i, 0)),
            pl.BlockSpec((_RB, D), lambda i: (i, 0)),
        ],
        out_shape=[
            jax.ShapeDtypeStruct((N, D), jnp.float32),
            jax.ShapeDtypeStruct((2, N, D), jnp.float32),
            jax.ShapeDtypeStruct((N, D), jnp.float32),
        ],
    )(x, w_cat, b_cat, deg_t)


def _fin_body(hhp_ref, prop_ref, hi_ref, degt_ref, gw_ref,
              hhp0_ref, prop0_ref, hi0_ref, deg0_ref, out_ref):
    d0 = deg0_ref[0, 0] + deg0_ref[0, 1]
    dinv0 = jnp.where(d0 > 0, lax.rsqrt(d0), 0.0)
    hhp0 = jnp.maximum(hhp0_ref[0:1, :] - dinv0 * prop0_ref[0, 0:1, :], 0.0)
    hlp0 = jnp.maximum(dinv0 * prop0_ref[1, 0:1, :], 0.0)
    hi0 = hi0_ref[0:1, :]
    ga = jnp.sum(hhp0 * gw_ref[0:1, :]) + gw_ref[3, 0]
    gb = jnp.sum(hlp0 * gw_ref[1:2, :]) + gw_ref[3, 1]
    gc = jnp.sum(hi0 * gw_ref[2:3, :]) + gw_ref[3, 2]

    degb = degt_ref[...]
    deg = degb[:, 0:1] + degb[:, 1:2]
    dinv = jnp.where(deg > 0, lax.rsqrt(deg), 0.0)
    h_hp = jnp.maximum(hhp_ref[...] - dinv * prop_ref[0], 0.0)
    h_lp = jnp.maximum(dinv * prop_ref[1], 0.0)
    z = ga * h_hp + gb * h_lp + gc * hi_ref[...]
    m = jnp.max(z, axis=1, keepdims=True)
    zs = z - m
    out_ref[...] = zs - jnp.log(jnp.sum(jnp.exp(zs), axis=1, keepdims=True))


def _fin_call(h_hp, prop, h_i, deg_t, gw):
    return pl.pallas_call(
        _fin_body,
        grid=(N // _RB,),
        in_specs=[
            pl.BlockSpec((_RB, D), lambda i: (i, 0)),
            pl.BlockSpec((2, _RB, D), lambda i: (0, i, 0)),
            pl.BlockSpec((_RB, D), lambda i: (i, 0)),
            pl.BlockSpec((_RB, 2), lambda i: (i, 0)),
            pl.BlockSpec((8, D), lambda i: (0, 0)),
            pl.BlockSpec((8, D), lambda i: (0, 0)),
            pl.BlockSpec((2, 8, D), lambda i: (0, 0, 0)),
            pl.BlockSpec((8, D), lambda i: (0, 0)),
            pl.BlockSpec((8, 2), lambda i: (0, 0)),
        ],
        out_specs=pl.BlockSpec((_RB, D), lambda i: (i, 0)),
        out_shape=jax.ShapeDtypeStruct((N, D), jnp.float32),
    )(h_hp, prop, h_i, deg_t, gw, h_hp, prop, h_i, deg_t)


def kernel(x, edge_index, W_hp, b_hp, W_lp, b_lp, W_i, b_i,
           w_h, bh, w_l, bl, w_i, bi):
    src = edge_index[0]
    dst = edge_index[1]

    deg2 = _deg_kernel(dst)
    deg_t = jnp.transpose(deg2)[:N]

    w_cat = jnp.concatenate([W_hp, W_lp, W_i], axis=1)
    b_cat = jnp.zeros((8, 3 * D), jnp.float32).at[0].set(
        jnp.concatenate([b_hp, b_lp, b_i]))

    h_hp, hs, h_i = _mm_call(x, w_cat, b_cat, deg_t)

    hs_flat = hs.reshape(2 * N, D)
    src2 = jnp.concatenate([src, src + N])
    zrows = jnp.zeros((ZCHK, D), jnp.float32)

    prop = _prop_kernel(hs_flat, src2, dst, zrows)

    gw = (jnp.zeros((8, D), jnp.float32)
          .at[0].set(w_h[:, 0]).at[1].set(w_l[:, 0]).at[2].set(w_i[:, 0])
          .at[3, 0].set(bh[0]).at[3, 1].set(bl[0]).at[3, 2].set(bi[0]))

    return _fin_call(h_hp, prop, h_i, deg_t, gw)


# speedup vs baseline: 11.8789x; 1.0554x over previous
import functools

import jax
import jax.numpy as jnp
from jax import lax
from jax.experimental import pallas as pl
from jax.experimental.pallas import tpu as pltpu
from jax.experimental.pallas import tpu_sc as plsc

N = 10000
D = 128
NE = 320000
NPAD = 10240
CHK = 128
NCHNK = 160
EPT = NCHNK * CHK
NE_PAD = EPT * 16
NROW2 = NE_PAD // CHK
NBUF = 2
G = 16
DEG_ROWS = NROW2 // 32
ROWS_PER_TILE = NPAD // 16

_MESH = plsc.VectorSubcoreMesh(core_axis_name="c", subcore_axis_name="s")


@functools.partial(
    pl.kernel,
    out_type=jax.ShapeDtypeStruct((2, NPAD), jnp.float32),
    mesh=_MESH,
    scratch_types=[
        pltpu.VMEM((DEG_ROWS, CHK), jnp.int32),
        pltpu.VMEM((CHK,), jnp.float32),
        pltpu.VMEM((NPAD // 16,), jnp.float32),
        pltpu.VMEM_SHARED((NPAD,), jnp.float32),
        pltpu.SemaphoreType.DMA,
    ],
)
def _deg_kernel(dst2_hbm, out_hbm, di_v, ones_v, z_v, acc, sem):
    c = lax.axis_index("c")
    s = lax.axis_index("s")
    for i in range(CHK // 16):
        ones_v[pl.ds(i * 16, 16)] = jnp.full((16,), 1.0, jnp.float32)
    for i in range((NPAD // 16) // 16):
        z_v[pl.ds(i * 16, 16)] = jnp.zeros((16,), jnp.float32)
    strip = NPAD // 16
    pltpu.sync_copy(z_v, acc.at[pl.ds(s * strip, strip)])
    w = c * 16 + s
    pltpu.sync_copy(dst2_hbm.at[pl.ds(w * DEG_ROWS, DEG_ROWS)], di_v)
    plsc.subcore_barrier()

    def body(i, _):
        for b in range(8):
            pltpu.async_copy(ones_v, acc.at[di_v.at[i * 8 + b]], sem,
                             add=True)
        for b in range(8):
            pltpu.make_async_copy(ones_v, acc.at[di_v.at[0]], sem).wait()
        return ()

    lax.fori_loop(0, DEG_ROWS // 8, body, ())
    plsc.subcore_barrier()
    pltpu.sync_copy(acc.at[pl.ds(s * strip, strip)],
                    out_hbm.at[c, pl.ds(s * strip, strip)])


@functools.partial(
    pl.kernel,
    out_type=jax.ShapeDtypeStruct((2, NPAD, D), jnp.float32),
    mesh=_MESH,
    scratch_types=[
        pltpu.VMEM((G, CHK), jnp.int32),
        pltpu.VMEM((G, CHK), jnp.int32),
        [pltpu.VMEM((CHK, D), jnp.float32) for _ in range(NBUF)],
        pltpu.VMEM_SHARED((NPAD, D), jnp.float32),
        [pltpu.SemaphoreType.DMA for _ in range(NBUF)],
        [pltpu.SemaphoreType.DMA for _ in range(NBUF)],
    ],
)
def _prop_kernel(hs_hbm, src2_hbm, dst2_hbm, zrows_hbm, out_hbm,
                 si_v, di_v, rows, acc, sg, ss):
    c = lax.axis_index("c")
    s = lax.axis_index("s")
    row0 = s * ROWS_PER_TILE
    for q in range(ROWS_PER_TILE // CHK):
        pltpu.sync_copy(zrows_hbm, acc.at[pl.ds(row0 + q * CHK, CHK)])
    plsc.subcore_barrier()

    def gather(j, b):
        pltpu.async_copy(hs_hbm.at[si_v.at[j]], rows[b], sg[b])

    def wait_gather(b):
        pltpu.make_async_copy(hs_hbm.at[si_v.at[0]], rows[b], sg[b]).wait()

    def scat(j, b):
        pltpu.async_copy(rows[b], acc.at[di_v.at[j]], ss[b], add=True)

    def wait_scat(j, b):
        pltpu.make_async_copy(rows[b], acc.at[di_v.at[j]], ss[b]).wait()

    def group(g, _):
        pltpu.sync_copy(
            src2_hbm.at[pl.ds(c * NROW2 + s * NCHNK + g * G, G)], si_v)
        pltpu.sync_copy(dst2_hbm.at[pl.ds(s * NCHNK + g * G, G)], di_v)
        gather(0, 0)
        gather(1, 1)

        def inner(i, _):
            j = 2 * i
            wait_gather(0)
            scat(j, 0)
            wait_gather(1)
            scat(j + 1, 1)
            wait_scat(j, 0)
            gather(j + 2, 0)
            wait_scat(j + 1, 1)
            gather(j + 3, 1)
            return ()

        lax.fori_loop(0, G // 2 - 1, inner, ())
        wait_gather(0)
        scat(G - 2, 0)
        wait_gather(1)
        scat(G - 1, 1)
        wait_scat(G - 2, 0)
        wait_scat(G - 1, 1)
        return ()

    lax.fori_loop(0, NCHNK // G, group, ())
    plsc.subcore_barrier()
    pltpu.sync_copy(acc.at[pl.ds(row0, ROWS_PER_TILE)],
                    out_hbm.at[c, pl.ds(row0, ROWS_PER_TILE)])


_RB = 2000


def _mm_body(x_ref, w_ref, b_ref, degt_ref, hhp_ref, hs_ref, hi_ref):
    xb = x_ref[...]
    acc = jnp.dot(xb, w_ref[...], preferred_element_type=jnp.float32)
    acc = acc + b_ref[0:1, :]
    degb = degt_ref[...]
    deg = degb[:, 0:1] + degb[:, 1:2]
    dinv = jnp.where(deg > 0, lax.rsqrt(deg), 0.0)
    h_hp = acc[:, 0:D]
    h_lp = acc[:, D:2 * D]
    h_i = acc[:, 2 * D:3 * D]
    hhp_ref[...] = h_hp
    hs_ref[0] = dinv * h_hp
    hs_ref[1] = dinv * h_lp
    hi_ref[...] = jnp.maximum(h_i, 0.0)


def _mm_call(x, w_cat, b_cat, deg_t):
    return pl.pallas_call(
        _mm_body,
        grid=(N // _RB,),
        in_specs=[
            pl.BlockSpec((_RB, D), lambda i: (i, 0)),
            pl.BlockSpec((D, 3 * D), lambda i: (0, 0)),
            pl.BlockSpec((8, 3 * D), lambda i: (0, 0)),
            pl.BlockSpec((_RB, 2), lambda i: (i, 0)),
        ],
        out_specs=[
            pl.BlockSpec((_RB, D), lambda i: (i, 0)),
            pl.BlockSpec((2, _RB, D), lambda i: (0, i, 0)),
            pl.BlockSpec((_RB, D), lambda i: (i, 0)),
        ],
        out_shape=[
            jax.ShapeDtypeStruct((N, D), jnp.float32),
            jax.ShapeDtypeStruct((2, N, D), jnp.float32),
            jax.ShapeDtypeStruct((N, D), jnp.float32),
        ],
    )(x, w_cat, b_cat, deg_t)


def _fin_body(hhp_ref, prop_ref, hi_ref, degt_ref, gw_ref,
              hhp0_ref, prop0_ref, hi0_ref, deg0_ref, out_ref):
    d0 = deg0_ref[0, 0] + deg0_ref[0, 1]
    dinv0 = jnp.where(d0 > 0, lax.rsqrt(d0), 0.0)
    hhp0 = jnp.maximum(hhp0_ref[0:1, :] - dinv0 * prop0_ref[0, 0:1, :], 0.0)
    hlp0 = jnp.maximum(dinv0 * prop0_ref[1, 0:1, :], 0.0)
    hi0 = hi0_ref[0:1, :]
    ga = jnp.sum(hhp0 * gw_ref[0:1, :]) + gw_ref[3, 0]
    gb = jnp.sum(hlp0 * gw_ref[1:2, :]) + gw_ref[3, 1]
    gc = jnp.sum(hi0 * gw_ref[2:3, :]) + gw_ref[3, 2]

    degb = degt_ref[...]
    deg = degb[:, 0:1] + degb[:, 1:2]
    dinv = jnp.where(deg > 0, lax.rsqrt(deg), 0.0)
    h_hp = jnp.maximum(hhp_ref[...] - dinv * prop_ref[0], 0.0)
    h_lp = jnp.maximum(dinv * prop_ref[1], 0.0)
    z = ga * h_hp + gb * h_lp + gc * hi_ref[...]
    m = jnp.max(z, axis=1, keepdims=True)
    zs = z - m
    out_ref[...] = zs - jnp.log(jnp.sum(jnp.exp(zs), axis=1, keepdims=True))


def _fin_call(h_hp, prop, h_i, deg_t, gw):
    return pl.pallas_call(
        _fin_body,
        grid=(N // _RB,),
        in_specs=[
            pl.BlockSpec((_RB, D), lambda i: (i, 0)),
            pl.BlockSpec((2, _RB, D), lambda i: (0, i, 0)),
            pl.BlockSpec((_RB, D), lambda i: (i, 0)),
            pl.BlockSpec((_RB, 2), lambda i: (i, 0)),
            pl.BlockSpec((8, D), lambda i: (0, 0)),
            pl.BlockSpec((8, D), lambda i: (0, 0)),
            pl.BlockSpec((2, 8, D), lambda i: (0, 0, 0)),
            pl.BlockSpec((8, D), lambda i: (0, 0)),
            pl.BlockSpec((8, 2), lambda i: (0, 0)),
        ],
        out_specs=pl.BlockSpec((_RB, D), lambda i: (i, 0)),
        out_shape=jax.ShapeDtypeStruct((N, D), jnp.float32),
    )(h_hp, prop, h_i, deg_t, gw, h_hp, prop, h_i, deg_t)


def kernel(x, edge_index, W_hp, b_hp, W_lp, b_lp, W_i, b_i,
           w_h, bh, w_l, bl, w_i, bi):
    src = edge_index[0]
    dst = edge_index[1]

    pad = NE_PAD - NE
    src_pad = jnp.concatenate([src, jnp.zeros((pad,), jnp.int32)])
    dst_pad = jnp.concatenate([dst, jnp.full((pad,), N, jnp.int32)])
    src2 = jnp.concatenate([src_pad, src_pad + N]).reshape(2 * NROW2, CHK)
    dst2 = dst_pad.reshape(NROW2, CHK)

    deg2 = _deg_kernel(dst2)
    deg_t = jnp.transpose(deg2)[:N]

    w_cat = jnp.concatenate([W_hp, W_lp, W_i], axis=1)
    b_cat = jnp.zeros((8, 3 * D), jnp.float32).at[0].set(
        jnp.concatenate([b_hp, b_lp, b_i]))

    h_hp, hs, h_i = _mm_call(x, w_cat, b_cat, deg_t)

    hs_flat = hs.reshape(2 * N, D)
    zrows = jnp.zeros((CHK, D), jnp.float32)

    prop = _prop_kernel(hs_flat, src2, dst2, zrows)

    gw = (jnp.zeros((8, D), jnp.float32)
          .at[0].set(w_h[:, 0]).at[1].set(w_l[:, 0]).at[2].set(w_i[:, 0])
          .at[3, 0].set(bh[0]).at[3, 1].set(bl[0]).at[3, 2].set(bi[0]))

    return _fin_call(h_hp, prop, h_i, deg_t, gw)
